# Initial kernel scaffold; baseline (speedup 1.0000x reference)
import jax
import jax.numpy as jnp
from jax import lax
from jax.experimental import pallas as pl
from jax.experimental.pallas import tpu as pltpu
from jax.experimental.pallas import tpu_sc as plsc

DIM = 128
NV = 10000
NE = 50000
NPAD = 10240
SENT = NV
NNZ = 500000
CHUNK = 128
NW = 32
NCHUNK_W = 123
PER_W = NCHUNK_W * CHUNK
NNZ_PAD = NW * PER_W
STRIPE = NPAD // 16


def _elu(h):
    return jnp.where(h > 0, h, jnp.exp(h) - 1.0)


def _sc_mesh():
    return plsc.VectorSubcoreMesh(core_axis_name="c", subcore_axis_name="s")


def _sc_pass_counts(gid, scat, tab, z128, z16, ones_hbm):
    def body(gid_hbm, scat_hbm, tab_hbm, z128_hbm, z16_hbm, ones16_hbm,
             acc_out, cnt_out,
             acc_sh, cs_sh, cg_sh, idxg, idxs, rows, ones_b, sem):
        cid = lax.axis_index("c")
        sid = lax.axis_index("s")
        wid = sid * 2 + cid
        s0 = sid * STRIPE
        pltpu.sync_copy(z128_hbm, acc_sh.at[pl.ds(s0, STRIPE)])
        pltpu.sync_copy(z16_hbm, cs_sh.at[pl.ds(s0, STRIPE)])
        pltpu.sync_copy(z16_hbm, cg_sh.at[pl.ds(s0, STRIPE)])
        pltpu.sync_copy(ones16_hbm, ones_b)
        plsc.subcore_barrier()

        def chunk(c, carry):
            off = wid * PER_W + c * CHUNK
            pltpu.sync_copy(gid_hbm.at[pl.ds(off, CHUNK)], idxg)
            pltpu.sync_copy(scat_hbm.at[pl.ds(off, CHUNK)], idxs)
            pltpu.async_copy(tab_hbm.at[idxg], rows, sem).wait()
            pltpu.sync_copy(rows, acc_sh.at[idxs], add=True)
            pltpu.sync_copy(ones_b, cs_sh.at[idxs], add=True)
            pltpu.sync_copy(ones_b, cg_sh.at[idxg], add=True)
            return carry

        lax.fori_loop(0, NCHUNK_W, chunk, 0)
        plsc.subcore_barrier()
        pltpu.sync_copy(acc_sh.at[pl.ds(s0, STRIPE)],
                        acc_out.at[cid, pl.ds(s0, STRIPE)])
        pltpu.sync_copy(cs_sh.at[pl.ds(s0, STRIPE)],
                        cnt_out.at[cid, 0, pl.ds(s0, STRIPE)])
        pltpu.sync_copy(cg_sh.at[pl.ds(s0, STRIPE)],
                        cnt_out.at[cid, 1, pl.ds(s0, STRIPE)])

    f = pl.kernel(
        body,
        out_type=(jax.ShapeDtypeStruct((2, NPAD, DIM), jnp.float32),
                  jax.ShapeDtypeStruct((2, 2, NPAD, 16), jnp.float32)),
        mesh=_sc_mesh(),
        scratch_types=(
            pltpu.VMEM_SHARED((NPAD, DIM), jnp.float32),
            pltpu.VMEM_SHARED((NPAD, 16), jnp.float32),
            pltpu.VMEM_SHARED((NPAD, 16), jnp.float32),
            pltpu.VMEM((CHUNK,), jnp.int32),
            pltpu.VMEM((CHUNK,), jnp.int32),
            pltpu.VMEM((CHUNK, DIM), jnp.float32),
            pltpu.VMEM((CHUNK, 16), jnp.float32),
            pltpu.SemaphoreType.DMA,
        ),
    )
    return f(gid, scat, tab, z128, z16, ones_hbm)


def _sc_pass_plain(gid, scat, tab, z128):
    def body(gid_hbm, scat_hbm, tab_hbm, z128_hbm,
             acc_out,
             acc_sh, idxg, idxs, rows, sem):
        cid = lax.axis_index("c")
        sid = lax.axis_index("s")
        wid = sid * 2 + cid
        s0 = sid * STRIPE
        pltpu.sync_copy(z128_hbm, acc_sh.at[pl.ds(s0, STRIPE)])
        plsc.subcore_barrier()

        def chunk(c, carry):
            off = wid * PER_W + c * CHUNK
            pltpu.sync_copy(gid_hbm.at[pl.ds(off, CHUNK)], idxg)
            pltpu.sync_copy(scat_hbm.at[pl.ds(off, CHUNK)], idxs)
            pltpu.async_copy(tab_hbm.at[idxg], rows, sem).wait()
            pltpu.sync_copy(rows, acc_sh.at[idxs], add=True)
            return carry

        lax.fori_loop(0, NCHUNK_W, chunk, 0)
        plsc.subcore_barrier()
        pltpu.sync_copy(acc_sh.at[pl.ds(s0, STRIPE)],
                        acc_out.at[cid, pl.ds(s0, STRIPE)])

    f = pl.kernel(
        body,
        out_type=jax.ShapeDtypeStruct((2, NPAD, DIM), jnp.float32),
        mesh=_sc_mesh(),
        scratch_types=(
            pltpu.VMEM_SHARED((NPAD, DIM), jnp.float32),
            pltpu.VMEM((CHUNK,), jnp.int32),
            pltpu.VMEM((CHUNK,), jnp.int32),
            pltpu.VMEM((CHUNK, DIM), jnp.float32),
            pltpu.SemaphoreType.DMA,
        ),
    )
    return f(gid, scat, tab, z128)


_BT = 512
_BN = 500


def _tc_edge_table(acc2, cnt, E, Wn):
    def body(acc_ref, cnt_ref, e_ref, wn_ref, out_ref):
        acc = acc_ref[0] + acc_ref[1]
        deg = jnp.maximum(cnt_ref[0, 0, :, 0] + cnt_ref[1, 0, :, 0], 1.0)
        em = acc / deg[:, None]
        h = jnp.dot(em, wn_ref[...], preferred_element_type=jnp.float32)
        out_ref[...] = em + e_ref[...] + _elu(h)

    return pl.pallas_call(
        body,
        grid=(NPAD // _BT,),
        in_specs=[
            pl.BlockSpec((2, _BT, DIM), lambda i: (0, i, 0)),
            pl.BlockSpec((2, 2, _BT, 16), lambda i: (0, 0, i, 0)),
            pl.BlockSpec((_BT, DIM), lambda i: (i, 0)),
            pl.BlockSpec((DIM, DIM), lambda i: (0, 0)),
        ],
        out_specs=pl.BlockSpec((_BT, DIM), lambda i: (i, 0)),
        out_shape=jax.ShapeDtypeStruct((NPAD, DIM), jnp.float32),
    )(acc2, cnt, E, Wn)


def _tc_node_out(accv2, cnt, x, E, Wn, We):
    def body(accv_ref, cnt_ref, x_ref, ehi_ref, wn_ref, we_ref, out_ref):
        xv = x_ref[...]
        h = jnp.dot(xv, wn_ref[...], preferred_element_type=jnp.float32)
        selfrow = xv + ehi_ref[...] + _elu(h)
        m = accv_ref[0] + accv_ref[1] + selfrow
        deg = cnt_ref[0, 1, :, 0] + cnt_ref[1, 1, :, 0] + 1.0
        g = jnp.dot(m / deg[:, None], we_ref[...],
                    preferred_element_type=jnp.float32)
        out_ref[...] = xv + _elu(g)

    return pl.pallas_call(
        body,
        grid=(NV // _BN,),
        in_specs=[
            pl.BlockSpec((2, _BN, DIM), lambda i: (0, i, 0)),
            pl.BlockSpec((2, 2, _BN, 16), lambda i: (0, 0, i, 0)),
            pl.BlockSpec((_BN, DIM), lambda i: (i, 0)),
            pl.BlockSpec((_BN, DIM), lambda i: (i + NE // _BN, 0)),
            pl.BlockSpec((DIM, DIM), lambda i: (0, 0)),
            pl.BlockSpec((DIM, DIM), lambda i: (0, 0)),
        ],
        out_specs=pl.BlockSpec((_BN, DIM), lambda i: (i, 0)),
        out_shape=jax.ShapeDtypeStruct((NV, DIM), jnp.float32),
    )(accv2, cnt, x, E, Wn, We)


_TOPB = NV // _BN


def _tc_edge_out_pair(lo_a, lo_b, E_a, E_b):
    def body(la_ref, lb_ref, ea_ref, eb_ref, out_ref):
        i = pl.program_id(0)

        @pl.when(i < _TOPB)
        def _():
            out_ref[...] = la_ref[...] + lb_ref[...]

        @pl.when(i >= _TOPB)
        def _():
            out_ref[...] = ea_ref[...] + eb_ref[...]

    lo_spec = pl.BlockSpec((_BN, DIM), lambda i: (jnp.minimum(i, _TOPB - 1), 0))
    e_spec = pl.BlockSpec((_BN, DIM), lambda i: (jnp.maximum(i, _TOPB), 0))
    return pl.pallas_call(
        body,
        grid=(NE // _BN,),
        in_specs=[lo_spec, lo_spec, e_spec, e_spec],
        out_specs=pl.BlockSpec((_BN, DIM), lambda i: (i, 0)),
        out_shape=jax.ShapeDtypeStruct((NE, DIM), jnp.float32),
    )(lo_a, lo_b, E_a, E_b)


def _tc_edge_out_single(lo, E):
    def body(lo_ref, e_ref, out_ref):
        i = pl.program_id(0)

        @pl.when(i < _TOPB)
        def _():
            out_ref[...] = lo_ref[...]

        @pl.when(i >= _TOPB)
        def _():
            out_ref[...] = e_ref[...]

    return pl.pallas_call(
        body,
        grid=(NE // _BN,),
        in_specs=[
            pl.BlockSpec((_BN, DIM), lambda i: (jnp.minimum(i, _TOPB - 1), 0)),
            pl.BlockSpec((_BN, DIM), lambda i: (jnp.maximum(i, _TOPB), 0)),
        ],
        out_specs=pl.BlockSpec((_BN, DIM), lambda i: (i, 0)),
        out_shape=jax.ShapeDtypeStruct((NE, DIM), jnp.float32),
    )(lo, E)


def kernel(hyperedge_index_diag, hyperedge_index_proc, hyperedge_index_med,
           X_diag, X_proc, X_med, E_diag, E_proc, E_med,
           Wn_diag, Wn_proc, Wn_med, We_diag, We_proc, We_med):
    z128 = jnp.zeros((STRIPE, DIM), jnp.float32)
    z16 = jnp.zeros((STRIPE, 16), jnp.float32)
    ones16 = jnp.ones((CHUNK, 16), jnp.float32)

    def one_type(hidx, x, e, Wn, We):
        pad = jnp.full((2, NNZ_PAD - NNZ), SENT, jnp.int32)
        idx = jnp.concatenate([hidx, pad], axis=1)
        nid, eid = idx[0], idx[1]
        x_pad = jnp.zeros((NPAD, DIM), jnp.float32).at[:NV].set(x)
        acc2, cnt = _sc_pass_counts(nid, eid, x_pad, z128, z16, ones16)
        table = _tc_edge_table(acc2, cnt, e, Wn)
        accv2 = _sc_pass_plain(eid, nid, table, z128)
        node_out = _tc_node_out(accv2, cnt, x, e, Wn, We)
        return node_out, table

    Xd, lo_d = one_type(hyperedge_index_diag, X_diag, E_diag, Wn_diag, We_diag)
    Xp, lo_p = one_type(hyperedge_index_proc, X_proc, E_proc, Wn_proc, We_proc)
    Xm, lo_m = one_type(hyperedge_index_med, X_med, E_med, Wn_med, We_med)
    E_dp = _tc_edge_out_pair(lo_d, lo_p, E_diag, E_proc)
    E_m = _tc_edge_out_single(lo_m, E_med)
    return (Xd, Xp, Xm, E_dp, E_m)


# speedup vs baseline: 3.3461x; 3.3461x over previous
import jax
import jax.numpy as jnp
from jax import lax
from jax.experimental import pallas as pl
from jax.experimental.pallas import tpu as pltpu
from jax.experimental.pallas import tpu_sc as plsc

DIM = 128
NV = 10000
NE = 50000
NPAD = 10240
SENT = NV
NNZ = 500000
CHUNK = 128
NW = 32
NCHUNK_W = 123
PER_W = NCHUNK_W * CHUNK
NNZ_PAD = NW * PER_W
STRIPE = NPAD // 16


def _elu(h):
    return jnp.where(h > 0, h, jnp.exp(h) - 1.0)


def _sc_mesh():
    return plsc.VectorSubcoreMesh(core_axis_name="c", subcore_axis_name="s")


_NT = 3
_CNTROWS = _NT * NPAD
_CSTRIPE = _CNTROWS // 16
_NCHUNK_CNT = _NT * NCHUNK_W


def _sc_count_one(ids, z16, ones16):
    def body(ids_hbm, z16_hbm, ones16_hbm,
             cnt_out,
             cnt_sh, idxb, ones_b):
        cid = lax.axis_index("c")
        sid = lax.axis_index("s")
        wid = sid * 2 + cid
        s0 = sid * _CSTRIPE
        pltpu.sync_copy(z16_hbm, cnt_sh.at[pl.ds(s0, _CSTRIPE)])
        pltpu.sync_copy(ones16_hbm, ones_b)
        plsc.subcore_barrier()

        def chunk(c, carry):
            off = wid * (_NCHUNK_CNT * CHUNK) + c * CHUNK
            pltpu.sync_copy(ids_hbm.at[pl.ds(off, CHUNK)], idxb)
            pltpu.sync_copy(ones_b, cnt_sh.at[idxb], add=True)
            return carry

        lax.fori_loop(0, _NCHUNK_CNT, chunk, 0)
        plsc.subcore_barrier()
        pltpu.sync_copy(cnt_sh.at[pl.ds(s0, _CSTRIPE)],
                        cnt_out.at[cid, pl.ds(s0, _CSTRIPE)])

    f = pl.kernel(
        body,
        out_type=jax.ShapeDtypeStruct((2, _CNTROWS, 16), jnp.float32),
        mesh=_sc_mesh(),
        scratch_types=(
            pltpu.VMEM_SHARED((_CNTROWS, 16), jnp.float32),
            pltpu.VMEM((CHUNK,), jnp.int32),
            pltpu.VMEM((CHUNK, 16), jnp.float32),
        ),
    )
    return f(ids, z16, ones16)


def _sc_pass_plain(gid, scat, tab, z128):
    def body(gid_hbm, scat_hbm, tab_hbm, z128_hbm,
             acc_out,
             acc_sh, idxg, idxs, rows, sem):
        cid = lax.axis_index("c")
        sid = lax.axis_index("s")
        wid = sid * 2 + cid
        s0 = sid * STRIPE
        pltpu.sync_copy(z128_hbm, acc_sh.at[pl.ds(s0, STRIPE)])
        plsc.subcore_barrier()

        def chunk(c, carry):
            off = wid * PER_W + c * CHUNK
            pltpu.sync_copy(gid_hbm.at[pl.ds(off, CHUNK)], idxg)
            pltpu.sync_copy(scat_hbm.at[pl.ds(off, CHUNK)], idxs)
            pltpu.async_copy(tab_hbm.at[idxg], rows, sem).wait()
            pltpu.sync_copy(rows, acc_sh.at[idxs], add=True)
            return carry

        lax.fori_loop(0, NCHUNK_W, chunk, 0)
        plsc.subcore_barrier()
        pltpu.sync_copy(acc_sh.at[pl.ds(s0, STRIPE)],
                        acc_out.at[cid, pl.ds(s0, STRIPE)])

    f = pl.kernel(
        body,
        out_type=jax.ShapeDtypeStruct((2, NPAD, DIM), jnp.float32),
        mesh=_sc_mesh(),
        scratch_types=(
            pltpu.VMEM_SHARED((NPAD, DIM), jnp.float32),
            pltpu.VMEM((CHUNK,), jnp.int32),
            pltpu.VMEM((CHUNK,), jnp.int32),
            pltpu.VMEM((CHUNK, DIM), jnp.float32),
            pltpu.SemaphoreType.DMA,
        ),
    )
    return f(gid, scat, tab, z128)


_BT = 512
_BN = 400


def _tc_edge_table(acc2, deg_e, E, Wn):
    def body(acc_ref, deg_ref, e_ref, wn_ref, out_ref):
        acc = acc_ref[0] + acc_ref[1]
        em = acc / deg_ref[...]
        h = jnp.dot(em, wn_ref[...], preferred_element_type=jnp.float32)
        out_ref[...] = em + e_ref[...] + _elu(h)

    return pl.pallas_call(
        body,
        grid=(NPAD // _BT,),
        in_specs=[
            pl.BlockSpec((2, _BT, DIM), lambda i: (0, i, 0)),
            pl.BlockSpec((_BT, 1), lambda i: (i, 0)),
            pl.BlockSpec((_BT, DIM), lambda i: (i, 0)),
            pl.BlockSpec((DIM, DIM), lambda i: (0, 0)),
        ],
        out_specs=pl.BlockSpec((_BT, DIM), lambda i: (i, 0)),
        out_shape=jax.ShapeDtypeStruct((NPAD, DIM), jnp.float32),
    )(acc2, deg_e, E, Wn)


def _tc_node_out(accv2, deg_v, x, E, Wn, We):
    def body(accv_ref, deg_ref, x_ref, ehi_ref, wn_ref, we_ref, out_ref):
        xv = x_ref[...]
        h = jnp.dot(xv, wn_ref[...], preferred_element_type=jnp.float32)
        selfrow = xv + ehi_ref[...] + _elu(h)
        m = accv_ref[0] + accv_ref[1] + selfrow
        g = jnp.dot(m / deg_ref[...], we_ref[...],
                    preferred_element_type=jnp.float32)
        out_ref[...] = xv + _elu(g)

    return pl.pallas_call(
        body,
        grid=(NV // _BN,),
        in_specs=[
            pl.BlockSpec((2, _BN, DIM), lambda i: (0, i, 0)),
            pl.BlockSpec((_BN, 1), lambda i: (i, 0)),
            pl.BlockSpec((_BN, DIM), lambda i: (i, 0)),
            pl.BlockSpec((_BN, DIM), lambda i: (i + NE // _BN, 0)),
            pl.BlockSpec((DIM, DIM), lambda i: (0, 0)),
            pl.BlockSpec((DIM, DIM), lambda i: (0, 0)),
        ],
        out_specs=pl.BlockSpec((_BN, DIM), lambda i: (i, 0)),
        out_shape=jax.ShapeDtypeStruct((NV, DIM), jnp.float32),
    )(accv2, deg_v, x, E, Wn, We)


_TOPB = NV // _BN


def _tc_edge_out_pair(lo_a, lo_b, E_a, E_b):
    def body(la_ref, lb_ref, ea_ref, eb_ref, out_ref):
        i = pl.program_id(0)

        @pl.when(i < _TOPB)
        def _():
            out_ref[...] = la_ref[...] + lb_ref[...]

        @pl.when(i >= _TOPB)
        def _():
            out_ref[...] = ea_ref[...] + eb_ref[...]

    lo_spec = pl.BlockSpec((_BN, DIM), lambda i: (jnp.minimum(i, _TOPB - 1), 0))
    e_spec = pl.BlockSpec((_BN, DIM), lambda i: (jnp.maximum(i, _TOPB), 0))
    return pl.pallas_call(
        body,
        grid=(NE // _BN,),
        in_specs=[lo_spec, lo_spec, e_spec, e_spec],
        out_specs=pl.BlockSpec((_BN, DIM), lambda i: (i, 0)),
        out_shape=jax.ShapeDtypeStruct((NE, DIM), jnp.float32),
    )(lo_a, lo_b, E_a, E_b)


def _tc_edge_out_single(lo, E):
    def body(lo_ref, e_ref, out_ref):
        i = pl.program_id(0)

        @pl.when(i < _TOPB)
        def _():
            out_ref[...] = lo_ref[...]

        @pl.when(i >= _TOPB)
        def _():
            out_ref[...] = e_ref[...]

    return pl.pallas_call(
        body,
        grid=(NE // _BN,),
        in_specs=[
            pl.BlockSpec((_BN, DIM), lambda i: (jnp.minimum(i, _TOPB - 1), 0)),
            pl.BlockSpec((_BN, DIM), lambda i: (jnp.maximum(i, _TOPB), 0)),
        ],
        out_specs=pl.BlockSpec((_BN, DIM), lambda i: (i, 0)),
        out_shape=jax.ShapeDtypeStruct((NE, DIM), jnp.float32),
    )(lo, E)


def kernel(hyperedge_index_diag, hyperedge_index_proc, hyperedge_index_med,
           X_diag, X_proc, X_med, E_diag, E_proc, E_med,
           Wn_diag, Wn_proc, Wn_med, We_diag, We_proc, We_med):
    z128 = jnp.zeros((STRIPE, DIM), jnp.float32)
    z16 = jnp.zeros((_CSTRIPE, 16), jnp.float32)
    ones16 = jnp.ones((CHUNK, 16), jnp.float32)

    pad = jnp.full((2, NNZ_PAD - NNZ), SENT, jnp.int32)
    hidxs = [jnp.concatenate([h, pad], axis=1)
             for h in (hyperedge_index_diag, hyperedge_index_proc,
                       hyperedge_index_med)]
    nid_all = jnp.concatenate([h[0] + t * NPAD for t, h in enumerate(hidxs)])
    eid_all = jnp.concatenate([h[1] + t * NPAD for t, h in enumerate(hidxs)])
    onesz = jnp.ones((_NT * NNZ_PAD,), jnp.float32)
    ce = jax.ops.segment_sum(onesz, eid_all, num_segments=_CNTROWS)
    cv = jax.ops.segment_sum(onesz, nid_all, num_segments=_CNTROWS)
    deg_e_all = jnp.maximum(ce, 1.0)[:, None]
    deg_v_all = (cv + 1.0)[:, None]

    def one_type(t, x, e, Wn, We):
        nid, eid = hidxs[t][0], hidxs[t][1]
        deg_e = lax.slice_in_dim(deg_e_all, t * NPAD, (t + 1) * NPAD, axis=0)
        deg_v = lax.slice_in_dim(deg_v_all, t * NPAD, (t + 1) * NPAD, axis=0)
        x_pad = jnp.zeros((NPAD, DIM), jnp.float32).at[:NV].set(x)
        acc2 = _sc_pass_plain(nid, eid, x_pad, z128)
        table = _tc_edge_table(acc2, deg_e, e, Wn)
        accv2 = _sc_pass_plain(eid, nid, table, z128)
        node_out = _tc_node_out(accv2, deg_v, x, e, Wn, We)
        return node_out, table

    Xd, lo_d = one_type(0, X_diag, E_diag, Wn_diag, We_diag)
    Xp, lo_p = one_type(1, X_proc, E_proc, Wn_proc, We_proc)
    Xm, lo_m = one_type(2, X_med, E_med, Wn_med, We_med)
    E_dp = _tc_edge_out_pair(lo_d, lo_p, E_diag, E_proc)
    E_m = _tc_edge_out_single(lo_m, E_med)
    return (Xd, Xp, Xm, E_dp, E_m)


# speedup vs baseline: 4.0920x; 1.2229x over previous
import jax
import jax.numpy as jnp
from jax import lax
from jax.experimental import pallas as pl
from jax.experimental.pallas import tpu as pltpu
from jax.experimental.pallas import tpu_sc as plsc

DIM = 128
NV = 10000
NE = 50000
NPAD = 10240
SENT = NV
NNZ = 500000
CHUNK = 128
NW = 32
NCHUNK_W = 123
PER_W = NCHUNK_W * CHUNK
NNZ_PAD = NW * PER_W
STRIPE = NPAD // 16


def _elu(h):
    return jnp.where(h > 0, h, jnp.exp(h) - 1.0)


def _sc_mesh():
    return plsc.VectorSubcoreMesh(core_axis_name="c", subcore_axis_name="s")


_NT = 3
_NKIND = 2 * _NT


def _sc_counts(ids6, zs16, basis6):
    def body(ids_hbm, zs_hbm, basis_hbm,
             cnt_out,
             cnt_sh, idxb, src_b):
        cid = lax.axis_index("c")
        sid = lax.axis_index("s")
        wid = sid * 2 + cid
        s0 = sid * STRIPE
        pltpu.sync_copy(zs_hbm, cnt_sh.at[pl.ds(s0, STRIPE)])
        plsc.subcore_barrier()

        for k in range(_NKIND):
            pltpu.sync_copy(basis_hbm.at[k], src_b)

            def chunk(c, carry):
                off = wid * PER_W + c * CHUNK
                pltpu.sync_copy(ids_hbm.at[k, pl.ds(off, CHUNK)], idxb)
                pltpu.sync_copy(src_b, cnt_sh.at[idxb], add=True)
                return carry

            lax.fori_loop(0, NCHUNK_W, chunk, 0)

        plsc.subcore_barrier()
        pltpu.sync_copy(cnt_sh.at[pl.ds(s0, STRIPE)],
                        cnt_out.at[cid, pl.ds(s0, STRIPE)])

    f = pl.kernel(
        body,
        out_type=jax.ShapeDtypeStruct((2, NPAD, DIM), jnp.float32),
        mesh=_sc_mesh(),
        scratch_types=(
            pltpu.VMEM_SHARED((NPAD, DIM), jnp.float32),
            pltpu.VMEM((CHUNK,), jnp.int32),
            pltpu.VMEM((CHUNK, DIM), jnp.float32),
        ),
    )
    return f(ids6, zs16, basis6)


def _sc_pass_plain(gid, scat, tab, z128):
    def body(gid_hbm, scat_hbm, tab_hbm, z128_hbm,
             acc_out,
             acc_sh, idxg, idxs, rows, sem):
        cid = lax.axis_index("c")
        sid = lax.axis_index("s")
        wid = sid * 2 + cid
        s0 = sid * STRIPE
        pltpu.sync_copy(z128_hbm, acc_sh.at[pl.ds(s0, STRIPE)])
        plsc.subcore_barrier()

        def chunk(c, carry):
            off = wid * PER_W + c * CHUNK
            pltpu.sync_copy(gid_hbm.at[pl.ds(off, CHUNK)], idxg)
            pltpu.sync_copy(scat_hbm.at[pl.ds(off, CHUNK)], idxs)
            pltpu.async_copy(tab_hbm.at[idxg], rows, sem).wait()
            pltpu.sync_copy(rows, acc_sh.at[idxs], add=True)
            return carry

        lax.fori_loop(0, NCHUNK_W, chunk, 0)
        plsc.subcore_barrier()
        pltpu.sync_copy(acc_sh.at[pl.ds(s0, STRIPE)],
                        acc_out.at[cid, pl.ds(s0, STRIPE)])

    f = pl.kernel(
        body,
        out_type=jax.ShapeDtypeStruct((2, NPAD, DIM), jnp.float32),
        mesh=_sc_mesh(),
        scratch_types=(
            pltpu.VMEM_SHARED((NPAD, DIM), jnp.float32),
            pltpu.VMEM((CHUNK,), jnp.int32),
            pltpu.VMEM((CHUNK,), jnp.int32),
            pltpu.VMEM((CHUNK, DIM), jnp.float32),
            pltpu.SemaphoreType.DMA,
        ),
    )
    return f(gid, scat, tab, z128)


_BT = 512
_BN = 400


def _tc_edge_table(acc2, deg_e, E, Wn):
    def body(acc_ref, deg_ref, e_ref, wn_ref, out_ref):
        acc = acc_ref[0] + acc_ref[1]
        em = acc / deg_ref[...]
        h = jnp.dot(em, wn_ref[...], preferred_element_type=jnp.float32)
        out_ref[...] = em + e_ref[...] + _elu(h)

    return pl.pallas_call(
        body,
        grid=(NPAD // _BT,),
        in_specs=[
            pl.BlockSpec((2, _BT, DIM), lambda i: (0, i, 0)),
            pl.BlockSpec((_BT, 1), lambda i: (i, 0)),
            pl.BlockSpec((_BT, DIM), lambda i: (i, 0)),
            pl.BlockSpec((DIM, DIM), lambda i: (0, 0)),
        ],
        out_specs=pl.BlockSpec((_BT, DIM), lambda i: (i, 0)),
        out_shape=jax.ShapeDtypeStruct((NPAD, DIM), jnp.float32),
    )(acc2, deg_e, E, Wn)


def _tc_node_out(accv2, deg_v, x, E, Wn, We):
    def body(accv_ref, deg_ref, x_ref, ehi_ref, wn_ref, we_ref, out_ref):
        xv = x_ref[...]
        h = jnp.dot(xv, wn_ref[...], preferred_element_type=jnp.float32)
        selfrow = xv + ehi_ref[...] + _elu(h)
        m = accv_ref[0] + accv_ref[1] + selfrow
        g = jnp.dot(m / deg_ref[...], we_ref[...],
                    preferred_element_type=jnp.float32)
        out_ref[...] = xv + _elu(g)

    return pl.pallas_call(
        body,
        grid=(NV // _BN,),
        in_specs=[
            pl.BlockSpec((2, _BN, DIM), lambda i: (0, i, 0)),
            pl.BlockSpec((_BN, 1), lambda i: (i, 0)),
            pl.BlockSpec((_BN, DIM), lambda i: (i, 0)),
            pl.BlockSpec((_BN, DIM), lambda i: (i + NE // _BN, 0)),
            pl.BlockSpec((DIM, DIM), lambda i: (0, 0)),
            pl.BlockSpec((DIM, DIM), lambda i: (0, 0)),
        ],
        out_specs=pl.BlockSpec((_BN, DIM), lambda i: (i, 0)),
        out_shape=jax.ShapeDtypeStruct((NV, DIM), jnp.float32),
    )(accv2, deg_v, x, E, Wn, We)


_TOPB = NV // _BN


def _tc_edge_out_pair(lo_a, lo_b, E_a, E_b):
    def body(la_ref, lb_ref, ea_ref, eb_ref, out_ref):
        i = pl.program_id(0)

        @pl.when(i < _TOPB)
        def _():
            out_ref[...] = la_ref[...] + lb_ref[...]

        @pl.when(i >= _TOPB)
        def _():
            out_ref[...] = ea_ref[...] + eb_ref[...]

    lo_spec = pl.BlockSpec((_BN, DIM), lambda i: (jnp.minimum(i, _TOPB - 1), 0))
    e_spec = pl.BlockSpec((_BN, DIM), lambda i: (jnp.maximum(i, _TOPB), 0))
    return pl.pallas_call(
        body,
        grid=(NE // _BN,),
        in_specs=[lo_spec, lo_spec, e_spec, e_spec],
        out_specs=pl.BlockSpec((_BN, DIM), lambda i: (i, 0)),
        out_shape=jax.ShapeDtypeStruct((NE, DIM), jnp.float32),
    )(lo_a, lo_b, E_a, E_b)


def _tc_edge_out_single(lo, E):
    def body(lo_ref, e_ref, out_ref):
        i = pl.program_id(0)

        @pl.when(i < _TOPB)
        def _():
            out_ref[...] = lo_ref[...]

        @pl.when(i >= _TOPB)
        def _():
            out_ref[...] = e_ref[...]

    return pl.pallas_call(
        body,
        grid=(NE // _BN,),
        in_specs=[
            pl.BlockSpec((_BN, DIM), lambda i: (jnp.minimum(i, _TOPB - 1), 0)),
            pl.BlockSpec((_BN, DIM), lambda i: (jnp.maximum(i, _TOPB), 0)),
        ],
        out_specs=pl.BlockSpec((_BN, DIM), lambda i: (i, 0)),
        out_shape=jax.ShapeDtypeStruct((NE, DIM), jnp.float32),
    )(lo, E)


def kernel(hyperedge_index_diag, hyperedge_index_proc, hyperedge_index_med,
           X_diag, X_proc, X_med, E_diag, E_proc, E_med,
           Wn_diag, Wn_proc, Wn_med, We_diag, We_proc, We_med):
    z128 = jnp.zeros((STRIPE, DIM), jnp.float32)
    zs16 = jnp.zeros((STRIPE, DIM), jnp.float32)
    lane = jnp.arange(DIM)
    basis6 = jnp.stack(
        [(lane == 16 * k).astype(jnp.float32) for k in range(_NKIND)])
    basis6 = jnp.broadcast_to(basis6[:, None, :], (_NKIND, CHUNK, DIM))

    pad = jnp.full((2, NNZ_PAD - NNZ), SENT, jnp.int32)
    hidxs = [jnp.concatenate([h, pad], axis=1)
             for h in (hyperedge_index_diag, hyperedge_index_proc,
                       hyperedge_index_med)]
    ids6 = jnp.stack([hidxs[0][1], hidxs[1][1], hidxs[2][1],
                      hidxs[0][0], hidxs[1][0], hidxs[2][0]])
    cnt2 = _sc_counts(ids6, zs16, basis6)
    cnt = (cnt2[0].astype(jnp.float32) + cnt2[1].astype(jnp.float32))
    deg_e3 = jnp.maximum(cnt[:, 0:48:16], 1.0)
    deg_v3 = cnt[:, 48:96:16] + 1.0

    def one_type(t, x, e, Wn, We):
        nid, eid = hidxs[t][0], hidxs[t][1]
        deg_e = lax.slice_in_dim(deg_e3, t, t + 1, axis=1)
        deg_v = lax.slice_in_dim(deg_v3, t, t + 1, axis=1)
        x_pad = jnp.zeros((NPAD, DIM), jnp.float32).at[:NV].set(x)
        acc2 = _sc_pass_plain(nid, eid, x_pad, z128)
        table = _tc_edge_table(acc2, deg_e, e, Wn)
        accv2 = _sc_pass_plain(eid, nid, table, z128)
        node_out = _tc_node_out(accv2, deg_v, x, e, Wn, We)
        return node_out, table

    Xd, lo_d = one_type(0, X_diag, E_diag, Wn_diag, We_diag)
    Xp, lo_p = one_type(1, X_proc, E_proc, Wn_proc, We_proc)
    Xm, lo_m = one_type(2, X_med, E_med, Wn_med, We_med)
    E_dp = _tc_edge_out_pair(lo_d, lo_p, E_diag, E_proc)
    E_m = _tc_edge_out_single(lo_m, E_med)
    return (Xd, Xp, Xm, E_dp, E_m)


# speedup vs baseline: 5.2290x; 1.2779x over previous
import jax
import jax.numpy as jnp
from jax import lax
from jax.experimental import pallas as pl
from jax.experimental.pallas import tpu as pltpu
from jax.experimental.pallas import tpu_sc as plsc

DIM = 128
NV = 10000
NE = 50000
NPAD = 10240
SENT = NV
NNZ = 500000
CHUNK = 128
NW = 32
NCHUNK_W = 123
PER_W = NCHUNK_W * CHUNK
NNZ_PAD = NW * PER_W
STRIPE = NPAD // 16


def _elu(h):
    return jnp.where(h > 0, h, jnp.exp(h) - 1.0)


def _sc_mesh():
    return plsc.VectorSubcoreMesh(core_axis_name="c", subcore_axis_name="s")


_NT = 3
_NKIND = 2 * _NT


def _sc_counts(ids6, zs16, basis6):
    def body(ids_hbm, zs_hbm, basis_hbm,
             cnt_out,
             cnt_sh, idxb, src_b, sem0, sem1):
        cid = lax.axis_index("c")
        sid = lax.axis_index("s")
        wid = sid * 2 + cid
        s0 = sid * STRIPE
        base = wid * PER_W
        pltpu.sync_copy(zs_hbm, cnt_sh.at[pl.ds(s0, STRIPE)])
        plsc.subcore_barrier()

        for k in range(_NKIND):
            pltpu.sync_copy(basis_hbm.at[k], src_b)
            pltpu.async_copy(ids_hbm.at[k, pl.ds(base, CHUNK)],
                             idxb.at[0], sem0)

            def chunk(c, carry):
                p = lax.rem(c, 2)
                off_n = base + jnp.minimum(c + 1, NCHUNK_W - 1) * CHUNK

                def step(p_, pn, sp, sn):
                    pltpu.async_copy(ids_hbm.at[k, pl.ds(off_n, CHUNK)],
                                     idxb.at[pn], sn)
                    pltpu.make_async_copy(ids_hbm.at[k, pl.ds(base, CHUNK)],
                                          idxb.at[p_], sp).wait()
                    pltpu.sync_copy(src_b, cnt_sh.at[idxb.at[p_]], add=True)

                @pl.when(p == 0)
                def _():
                    step(0, 1, sem0, sem1)

                @pl.when(p == 1)
                def _():
                    step(1, 0, sem1, sem0)

                return carry

            lax.fori_loop(0, NCHUNK_W, chunk, 0)
            pf = NCHUNK_W % 2
            sp = sem1 if pf else sem0
            pltpu.make_async_copy(ids_hbm.at[k, pl.ds(base, CHUNK)],
                                  idxb.at[pf], sp).wait()

        plsc.subcore_barrier()
        pltpu.sync_copy(cnt_sh.at[pl.ds(s0, STRIPE)],
                        cnt_out.at[cid, pl.ds(s0, STRIPE)])

    f = pl.kernel(
        body,
        out_type=jax.ShapeDtypeStruct((2, NPAD, DIM), jnp.float32),
        mesh=_sc_mesh(),
        scratch_types=(
            pltpu.VMEM_SHARED((NPAD, DIM), jnp.float32),
            pltpu.VMEM((2, CHUNK), jnp.int32),
            pltpu.VMEM((CHUNK, DIM), jnp.float32),
            pltpu.SemaphoreType.DMA,
            pltpu.SemaphoreType.DMA,
        ),
    )
    return f(ids6, zs16, basis6)


def _sc_pass_plain(gid, scat, tab, z128):
    def body(gid_hbm, scat_hbm, tab_hbm, z128_hbm,
             acc_out,
             acc_sh, idxg, idxs, rows, gsem, semg0, semg1, sems0, sems1):
        cid = lax.axis_index("c")
        sid = lax.axis_index("s")
        wid = sid * 2 + cid
        s0 = sid * STRIPE
        base = wid * PER_W
        pltpu.sync_copy(z128_hbm, acc_sh.at[pl.ds(s0, STRIPE)])
        plsc.subcore_barrier()

        pltpu.async_copy(gid_hbm.at[pl.ds(base, CHUNK)], idxg.at[0], semg0)
        pltpu.async_copy(scat_hbm.at[pl.ds(base, CHUNK)], idxs.at[0], sems0)

        def chunk(c, carry):
            p = lax.rem(c, 2)
            off_n = base + jnp.minimum(c + 1, NCHUNK_W - 1) * CHUNK

            def prefetch(pn):
                sg = semg1 if pn else semg0
                ss = sems1 if pn else sems0
                pltpu.async_copy(gid_hbm.at[pl.ds(off_n, CHUNK)],
                                 idxg.at[pn], sg)
                pltpu.async_copy(scat_hbm.at[pl.ds(off_n, CHUNK)],
                                 idxs.at[pn], ss)

            def wait_cur(p_):
                sg = semg1 if p_ else semg0
                ss = sems1 if p_ else sems0
                pltpu.make_async_copy(gid_hbm.at[pl.ds(base, CHUNK)],
                                      idxg.at[p_], sg).wait()
                pltpu.make_async_copy(scat_hbm.at[pl.ds(base, CHUNK)],
                                      idxs.at[p_], ss).wait()
                pltpu.async_copy(tab_hbm.at[idxg.at[p_]], rows, gsem).wait()
                pltpu.sync_copy(rows, acc_sh.at[idxs.at[p_]], add=True)

            @pl.when(p == 0)
            def _():
                prefetch(1)
                wait_cur(0)

            @pl.when(p == 1)
            def _():
                prefetch(0)
                wait_cur(1)

            return carry

        lax.fori_loop(0, NCHUNK_W, chunk, 0)
        pf = lax.rem(NCHUNK_W, 2)

        @pl.when(pf == 0)
        def _():
            pltpu.make_async_copy(gid_hbm.at[pl.ds(base, CHUNK)],
                                  idxg.at[0], semg0).wait()
            pltpu.make_async_copy(scat_hbm.at[pl.ds(base, CHUNK)],
                                  idxs.at[0], sems0).wait()

        @pl.when(pf == 1)
        def _():
            pltpu.make_async_copy(gid_hbm.at[pl.ds(base, CHUNK)],
                                  idxg.at[1], semg1).wait()
            pltpu.make_async_copy(scat_hbm.at[pl.ds(base, CHUNK)],
                                  idxs.at[1], sems1).wait()

        plsc.subcore_barrier()
        pltpu.sync_copy(acc_sh.at[pl.ds(s0, STRIPE)],
                        acc_out.at[cid, pl.ds(s0, STRIPE)])

    f = pl.kernel(
        body,
        out_type=jax.ShapeDtypeStruct((2, NPAD, DIM), jnp.float32),
        mesh=_sc_mesh(),
        scratch_types=(
            pltpu.VMEM_SHARED((NPAD, DIM), jnp.float32),
            pltpu.VMEM((2, CHUNK), jnp.int32),
            pltpu.VMEM((2, CHUNK), jnp.int32),
            pltpu.VMEM((CHUNK, DIM), jnp.float32),
            pltpu.SemaphoreType.DMA,
            pltpu.SemaphoreType.DMA,
            pltpu.SemaphoreType.DMA,
            pltpu.SemaphoreType.DMA,
            pltpu.SemaphoreType.DMA,
        ),
    )
    return f(gid, scat, tab, z128)


_BT = 512
_BN = 400


def _tc_edge_table(acc2, deg_e, E, Wn):
    def body(acc_ref, deg_ref, e_ref, wn_ref, out_ref):
        acc = acc_ref[0] + acc_ref[1]
        em = acc / deg_ref[...]
        h = jnp.dot(em, wn_ref[...], preferred_element_type=jnp.float32)
        out_ref[...] = em + e_ref[...] + _elu(h)

    return pl.pallas_call(
        body,
        grid=(NPAD // _BT,),
        in_specs=[
            pl.BlockSpec((2, _BT, DIM), lambda i: (0, i, 0)),
            pl.BlockSpec((_BT, 1), lambda i: (i, 0)),
            pl.BlockSpec((_BT, DIM), lambda i: (i, 0)),
            pl.BlockSpec((DIM, DIM), lambda i: (0, 0)),
        ],
        out_specs=pl.BlockSpec((_BT, DIM), lambda i: (i, 0)),
        out_shape=jax.ShapeDtypeStruct((NPAD, DIM), jnp.float32),
    )(acc2, deg_e, E, Wn)


def _tc_node_out(accv2, deg_v, x, E, Wn, We):
    def body(accv_ref, deg_ref, x_ref, ehi_ref, wn_ref, we_ref, out_ref):
        xv = x_ref[...]
        h = jnp.dot(xv, wn_ref[...], preferred_element_type=jnp.float32)
        selfrow = xv + ehi_ref[...] + _elu(h)
        m = accv_ref[0] + accv_ref[1] + selfrow
        g = jnp.dot(m / deg_ref[...], we_ref[...],
                    preferred_element_type=jnp.float32)
        out_ref[...] = xv + _elu(g)

    return pl.pallas_call(
        body,
        grid=(NV // _BN,),
        in_specs=[
            pl.BlockSpec((2, _BN, DIM), lambda i: (0, i, 0)),
            pl.BlockSpec((_BN, 1), lambda i: (i, 0)),
            pl.BlockSpec((_BN, DIM), lambda i: (i, 0)),
            pl.BlockSpec((_BN, DIM), lambda i: (i + NE // _BN, 0)),
            pl.BlockSpec((DIM, DIM), lambda i: (0, 0)),
            pl.BlockSpec((DIM, DIM), lambda i: (0, 0)),
        ],
        out_specs=pl.BlockSpec((_BN, DIM), lambda i: (i, 0)),
        out_shape=jax.ShapeDtypeStruct((NV, DIM), jnp.float32),
    )(accv2, deg_v, x, E, Wn, We)


_TOPB = NV // _BN


def _tc_edge_out_pair(lo_a, lo_b, E_a, E_b):
    def body(la_ref, lb_ref, ea_ref, eb_ref, out_ref):
        i = pl.program_id(0)

        @pl.when(i < _TOPB)
        def _():
            out_ref[...] = la_ref[...] + lb_ref[...]

        @pl.when(i >= _TOPB)
        def _():
            out_ref[...] = ea_ref[...] + eb_ref[...]

    lo_spec = pl.BlockSpec((_BN, DIM), lambda i: (jnp.minimum(i, _TOPB - 1), 0))
    e_spec = pl.BlockSpec((_BN, DIM), lambda i: (jnp.maximum(i, _TOPB), 0))
    return pl.pallas_call(
        body,
        grid=(NE // _BN,),
        in_specs=[lo_spec, lo_spec, e_spec, e_spec],
        out_specs=pl.BlockSpec((_BN, DIM), lambda i: (i, 0)),
        out_shape=jax.ShapeDtypeStruct((NE, DIM), jnp.float32),
    )(lo_a, lo_b, E_a, E_b)


def _tc_edge_out_single(lo, E):
    def body(lo_ref, e_ref, out_ref):
        i = pl.program_id(0)

        @pl.when(i < _TOPB)
        def _():
            out_ref[...] = lo_ref[...]

        @pl.when(i >= _TOPB)
        def _():
            out_ref[...] = e_ref[...]

    return pl.pallas_call(
        body,
        grid=(NE // _BN,),
        in_specs=[
            pl.BlockSpec((_BN, DIM), lambda i: (jnp.minimum(i, _TOPB - 1), 0)),
            pl.BlockSpec((_BN, DIM), lambda i: (jnp.maximum(i, _TOPB), 0)),
        ],
        out_specs=pl.BlockSpec((_BN, DIM), lambda i: (i, 0)),
        out_shape=jax.ShapeDtypeStruct((NE, DIM), jnp.float32),
    )(lo, E)


def kernel(hyperedge_index_diag, hyperedge_index_proc, hyperedge_index_med,
           X_diag, X_proc, X_med, E_diag, E_proc, E_med,
           Wn_diag, Wn_proc, Wn_med, We_diag, We_proc, We_med):
    z128 = jnp.zeros((STRIPE, DIM), jnp.float32)
    zs16 = jnp.zeros((STRIPE, DIM), jnp.float32)
    lane = jnp.arange(DIM)
    basis6 = jnp.stack(
        [(lane == 16 * k).astype(jnp.float32) for k in range(_NKIND)])
    basis6 = jnp.broadcast_to(basis6[:, None, :], (_NKIND, CHUNK, DIM))

    pad = jnp.full((2, NNZ_PAD - NNZ), SENT, jnp.int32)
    hidxs = [jnp.concatenate([h, pad], axis=1)
             for h in (hyperedge_index_diag, hyperedge_index_proc,
                       hyperedge_index_med)]
    ids6 = jnp.stack([hidxs[0][1], hidxs[1][1], hidxs[2][1],
                      hidxs[0][0], hidxs[1][0], hidxs[2][0]])
    cnt2 = _sc_counts(ids6, zs16, basis6)
    cnt = (cnt2[0].astype(jnp.float32) + cnt2[1].astype(jnp.float32))
    deg_e3 = jnp.maximum(cnt[:, 0:48:16], 1.0)
    deg_v3 = cnt[:, 48:96:16] + 1.0

    def one_type(t, x, e, Wn, We):
        nid, eid = hidxs[t][0], hidxs[t][1]
        deg_e = lax.slice_in_dim(deg_e3, t, t + 1, axis=1)
        deg_v = lax.slice_in_dim(deg_v3, t, t + 1, axis=1)
        x_pad = jnp.zeros((NPAD, DIM), jnp.float32).at[:NV].set(x)
        acc2 = _sc_pass_plain(nid, eid, x_pad, z128)
        table = _tc_edge_table(acc2, deg_e, e, Wn)
        accv2 = _sc_pass_plain(eid, nid, table, z128)
        node_out = _tc_node_out(accv2, deg_v, x, e, Wn, We)
        return node_out, table

    Xd, lo_d = one_type(0, X_diag, E_diag, Wn_diag, We_diag)
    Xp, lo_p = one_type(1, X_proc, E_proc, Wn_proc, We_proc)
    Xm, lo_m = one_type(2, X_med, E_med, Wn_med, We_med)
    E_dp = _tc_edge_out_pair(lo_d, lo_p, E_diag, E_proc)
    E_m = _tc_edge_out_single(lo_m, E_med)
    return (Xd, Xp, Xm, E_dp, E_m)


# speedup vs baseline: 5.9977x; 1.1470x over previous
import jax
import jax.numpy as jnp
from jax import lax
from jax.experimental import pallas as pl
from jax.experimental.pallas import tpu as pltpu
from jax.experimental.pallas import tpu_sc as plsc

DIM = 128
NV = 10000
NE = 50000
NPAD = 10240
SENT = NV
NNZ = 500000
CHUNK = 128
NW = 32
NCHUNK_W = 123
PER_W = NCHUNK_W * CHUNK
NNZ_PAD = NW * PER_W
STRIPE = NPAD // 16


def _elu(h):
    return jnp.where(h > 0, h, jnp.exp(h) - 1.0)


def _sc_mesh():
    return plsc.VectorSubcoreMesh(core_axis_name="c", subcore_axis_name="s")


_NT = 3
_NKIND = 2 * _NT


def _sc_counts(ids6, zs16, basis6):
    def body(ids_hbm, zs_hbm, basis_hbm,
             cnt_out,
             cnt_sh, idxb, src_b, sem0, sem1):
        cid = lax.axis_index("c")
        sid = lax.axis_index("s")
        wid = sid * 2 + cid
        s0 = sid * STRIPE
        base = wid * PER_W
        pltpu.sync_copy(zs_hbm, cnt_sh.at[pl.ds(s0, STRIPE)])
        plsc.subcore_barrier()

        for k in range(_NKIND):
            pltpu.sync_copy(basis_hbm.at[k], src_b)
            pltpu.async_copy(ids_hbm.at[k, pl.ds(base, CHUNK)],
                             idxb.at[0], sem0)

            def chunk(c, carry):
                p = lax.rem(c, 2)
                off_n = base + jnp.minimum(c + 1, NCHUNK_W - 1) * CHUNK

                def step(p_, pn, sp, sn):
                    pltpu.async_copy(ids_hbm.at[k, pl.ds(off_n, CHUNK)],
                                     idxb.at[pn], sn)
                    pltpu.make_async_copy(ids_hbm.at[k, pl.ds(base, CHUNK)],
                                          idxb.at[p_], sp).wait()
                    pltpu.sync_copy(src_b, cnt_sh.at[idxb.at[p_]], add=True)

                @pl.when(p == 0)
                def _():
                    step(0, 1, sem0, sem1)

                @pl.when(p == 1)
                def _():
                    step(1, 0, sem1, sem0)

                return carry

            lax.fori_loop(0, NCHUNK_W, chunk, 0)
            pf = NCHUNK_W % 2
            sp = sem1 if pf else sem0
            pltpu.make_async_copy(ids_hbm.at[k, pl.ds(base, CHUNK)],
                                  idxb.at[pf], sp).wait()

        plsc.subcore_barrier()
        pltpu.sync_copy(cnt_sh.at[pl.ds(s0, STRIPE)],
                        cnt_out.at[cid, pl.ds(s0, STRIPE)])

    f = pl.kernel(
        body,
        out_type=jax.ShapeDtypeStruct((2, NPAD, DIM), jnp.float32),
        mesh=_sc_mesh(),
        scratch_types=(
            pltpu.VMEM_SHARED((NPAD, DIM), jnp.float32),
            pltpu.VMEM((2, CHUNK), jnp.int32),
            pltpu.VMEM((CHUNK, DIM), jnp.float32),
            pltpu.SemaphoreType.DMA,
            pltpu.SemaphoreType.DMA,
        ),
    )
    return f(ids6, zs16, basis6)


P_CHUNK = 64
P_NCHUNK = PER_W // P_CHUNK


def _sc_pass_plain(gid, scat, tab, z128):
    def body(gid_hbm, scat_hbm, tab_hbm, z128_hbm,
             acc_out,
             acc_sh, idxg, idxs, rows,
             g0, g1, i0, i1, i2, j0, j1, j2):
        cid = lax.axis_index("c")
        sid = lax.axis_index("s")
        wid = sid * 2 + cid
        s0 = sid * STRIPE
        base = wid * PER_W
        isems = (i0, i1, i2)
        jsems = (j0, j1, j2)
        gsems = (g0, g1)
        pltpu.sync_copy(z128_hbm, acc_sh.at[pl.ds(s0, STRIPE)])
        plsc.subcore_barrier()

        def fire_idx(slot, off):
            pltpu.async_copy(gid_hbm.at[pl.ds(off, P_CHUNK)],
                             idxg.at[slot], isems[slot])
            pltpu.async_copy(scat_hbm.at[pl.ds(off, P_CHUNK)],
                             idxs.at[slot], jsems[slot])

        def wait_idx(slot):
            pltpu.make_async_copy(gid_hbm.at[pl.ds(base, P_CHUNK)],
                                  idxg.at[slot], isems[slot]).wait()
            pltpu.make_async_copy(scat_hbm.at[pl.ds(base, P_CHUNK)],
                                  idxs.at[slot], jsems[slot]).wait()

        fire_idx(0, base)
        fire_idx(1, base + P_CHUNK)
        wait_idx(0)
        pltpu.async_copy(tab_hbm.at[idxg.at[0]], rows.at[0], g0)

        def chunk(c, carry):
            off_n2 = base + jnp.minimum(c + 2, P_NCHUNK - 1) * P_CHUNK
            for r in range(6):

                @pl.when(lax.rem(c, 6) == r)
                def _(r=r):
                    p, s_c, s1, s2 = r % 2, r % 3, (r + 1) % 3, (r + 2) % 3
                    fire_idx(s2, off_n2)
                    wait_idx(s1)
                    pltpu.async_copy(tab_hbm.at[idxg.at[s1]],
                                     rows.at[1 - p], gsems[1 - p])
                    pltpu.make_async_copy(tab_hbm.at[idxg.at[s_c]],
                                          rows.at[p], gsems[p]).wait()
                    pltpu.sync_copy(rows.at[p], acc_sh.at[idxs.at[s_c]],
                                    add=True)

            return carry

        lax.fori_loop(0, P_NCHUNK, chunk, 0)
        pltpu.make_async_copy(tab_hbm.at[idxg.at[0]],
                              rows.at[0], g0).wait()
        wait_idx(1)
        plsc.subcore_barrier()
        pltpu.sync_copy(acc_sh.at[pl.ds(s0, STRIPE)],
                        acc_out.at[cid, pl.ds(s0, STRIPE)])

    f = pl.kernel(
        body,
        out_type=jax.ShapeDtypeStruct((2, NPAD, DIM), jnp.float32),
        mesh=_sc_mesh(),
        scratch_types=(
            pltpu.VMEM_SHARED((NPAD, DIM), jnp.float32),
            pltpu.VMEM((3, P_CHUNK), jnp.int32),
            pltpu.VMEM((3, P_CHUNK), jnp.int32),
            pltpu.VMEM((2, P_CHUNK, DIM), jnp.float32),
            pltpu.SemaphoreType.DMA,
            pltpu.SemaphoreType.DMA,
            pltpu.SemaphoreType.DMA,
            pltpu.SemaphoreType.DMA,
            pltpu.SemaphoreType.DMA,
            pltpu.SemaphoreType.DMA,
            pltpu.SemaphoreType.DMA,
            pltpu.SemaphoreType.DMA,
        ),
    )
    return f(gid, scat, tab, z128)


_BT = 512
_BN = 400


def _tc_edge_table(acc2, deg_e, E, Wn):
    def body(acc_ref, deg_ref, e_ref, wn_ref, out_ref):
        acc = acc_ref[0] + acc_ref[1]
        em = acc / deg_ref[...]
        h = jnp.dot(em, wn_ref[...], preferred_element_type=jnp.float32)
        out_ref[...] = em + e_ref[...] + _elu(h)

    return pl.pallas_call(
        body,
        grid=(NPAD // _BT,),
        in_specs=[
            pl.BlockSpec((2, _BT, DIM), lambda i: (0, i, 0)),
            pl.BlockSpec((_BT, 1), lambda i: (i, 0)),
            pl.BlockSpec((_BT, DIM), lambda i: (i, 0)),
            pl.BlockSpec((DIM, DIM), lambda i: (0, 0)),
        ],
        out_specs=pl.BlockSpec((_BT, DIM), lambda i: (i, 0)),
        out_shape=jax.ShapeDtypeStruct((NPAD, DIM), jnp.float32),
    )(acc2, deg_e, E, Wn)


def _tc_node_out(accv2, deg_v, x, E, Wn, We):
    def body(accv_ref, deg_ref, x_ref, ehi_ref, wn_ref, we_ref, out_ref):
        xv = x_ref[...]
        h = jnp.dot(xv, wn_ref[...], preferred_element_type=jnp.float32)
        selfrow = xv + ehi_ref[...] + _elu(h)
        m = accv_ref[0] + accv_ref[1] + selfrow
        g = jnp.dot(m / deg_ref[...], we_ref[...],
                    preferred_element_type=jnp.float32)
        out_ref[...] = xv + _elu(g)

    return pl.pallas_call(
        body,
        grid=(NV // _BN,),
        in_specs=[
            pl.BlockSpec((2, _BN, DIM), lambda i: (0, i, 0)),
            pl.BlockSpec((_BN, 1), lambda i: (i, 0)),
            pl.BlockSpec((_BN, DIM), lambda i: (i, 0)),
            pl.BlockSpec((_BN, DIM), lambda i: (i + NE // _BN, 0)),
            pl.BlockSpec((DIM, DIM), lambda i: (0, 0)),
            pl.BlockSpec((DIM, DIM), lambda i: (0, 0)),
        ],
        out_specs=pl.BlockSpec((_BN, DIM), lambda i: (i, 0)),
        out_shape=jax.ShapeDtypeStruct((NV, DIM), jnp.float32),
    )(accv2, deg_v, x, E, Wn, We)


_TOPB = NV // _BN


def _tc_edge_out_pair(lo_a, lo_b, E_a, E_b):
    def body(la_ref, lb_ref, ea_ref, eb_ref, out_ref):
        i = pl.program_id(0)

        @pl.when(i < _TOPB)
        def _():
            out_ref[...] = la_ref[...] + lb_ref[...]

        @pl.when(i >= _TOPB)
        def _():
            out_ref[...] = ea_ref[...] + eb_ref[...]

    lo_spec = pl.BlockSpec((_BN, DIM), lambda i: (jnp.minimum(i, _TOPB - 1), 0))
    e_spec = pl.BlockSpec((_BN, DIM), lambda i: (jnp.maximum(i, _TOPB), 0))
    return pl.pallas_call(
        body,
        grid=(NE // _BN,),
        in_specs=[lo_spec, lo_spec, e_spec, e_spec],
        out_specs=pl.BlockSpec((_BN, DIM), lambda i: (i, 0)),
        out_shape=jax.ShapeDtypeStruct((NE, DIM), jnp.float32),
    )(lo_a, lo_b, E_a, E_b)


def _tc_edge_out_single(lo, E):
    def body(lo_ref, e_ref, out_ref):
        i = pl.program_id(0)

        @pl.when(i < _TOPB)
        def _():
            out_ref[...] = lo_ref[...]

        @pl.when(i >= _TOPB)
        def _():
            out_ref[...] = e_ref[...]

    return pl.pallas_call(
        body,
        grid=(NE // _BN,),
        in_specs=[
            pl.BlockSpec((_BN, DIM), lambda i: (jnp.minimum(i, _TOPB - 1), 0)),
            pl.BlockSpec((_BN, DIM), lambda i: (jnp.maximum(i, _TOPB), 0)),
        ],
        out_specs=pl.BlockSpec((_BN, DIM), lambda i: (i, 0)),
        out_shape=jax.ShapeDtypeStruct((NE, DIM), jnp.float32),
    )(lo, E)


def kernel(hyperedge_index_diag, hyperedge_index_proc, hyperedge_index_med,
           X_diag, X_proc, X_med, E_diag, E_proc, E_med,
           Wn_diag, Wn_proc, Wn_med, We_diag, We_proc, We_med):
    z128 = jnp.zeros((STRIPE, DIM), jnp.float32)
    zs16 = jnp.zeros((STRIPE, DIM), jnp.float32)
    lane = jnp.arange(DIM)
    basis6 = jnp.stack(
        [(lane == 16 * k).astype(jnp.float32) for k in range(_NKIND)])
    basis6 = jnp.broadcast_to(basis6[:, None, :], (_NKIND, CHUNK, DIM))

    pad = jnp.full((2, NNZ_PAD - NNZ), SENT, jnp.int32)
    hidxs = [jnp.concatenate([h, pad], axis=1)
             for h in (hyperedge_index_diag, hyperedge_index_proc,
                       hyperedge_index_med)]
    ids6 = jnp.stack([hidxs[0][1], hidxs[1][1], hidxs[2][1],
                      hidxs[0][0], hidxs[1][0], hidxs[2][0]])
    cnt2 = _sc_counts(ids6, zs16, basis6)
    cnt = (cnt2[0].astype(jnp.float32) + cnt2[1].astype(jnp.float32))
    deg_e3 = jnp.maximum(cnt[:, 0:48:16], 1.0)
    deg_v3 = cnt[:, 48:96:16] + 1.0

    def one_type(t, x, e, Wn, We):
        nid, eid = hidxs[t][0], hidxs[t][1]
        deg_e = lax.slice_in_dim(deg_e3, t, t + 1, axis=1)
        deg_v = lax.slice_in_dim(deg_v3, t, t + 1, axis=1)
        x_pad = jnp.zeros((NPAD, DIM), jnp.float32).at[:NV].set(x)
        acc2 = _sc_pass_plain(nid, eid, x_pad, z128)
        table = _tc_edge_table(acc2, deg_e, e, Wn)
        accv2 = _sc_pass_plain(eid, nid, table, z128)
        node_out = _tc_node_out(accv2, deg_v, x, e, Wn, We)
        return node_out, table

    Xd, lo_d = one_type(0, X_diag, E_diag, Wn_diag, We_diag)
    Xp, lo_p = one_type(1, X_proc, E_proc, Wn_proc, We_proc)
    Xm, lo_m = one_type(2, X_med, E_med, Wn_med, We_med)
    E_dp = _tc_edge_out_pair(lo_d, lo_p, E_diag, E_proc)
    E_m = _tc_edge_out_single(lo_m, E_med)
    return (Xd, Xp, Xm, E_dp, E_m)
